# baseline (speedup 1.0000x reference)
import functools

import jax
import jax.numpy as jnp
from jax.experimental import pallas as pl
from jax.experimental.pallas import tpu as pltpu


def _sae_kernel_body(C, BR, BC, K, GPC,
                     x_ref, we_ref, be_ref, wd_ref, bd_ref,
                     zpre_ref, zsp_ref, xrec_ref,
                     zscr_ref, acc_ref, thr_ref, m_ref):
    c = pl.program_id(1)
    GW = BC // GPC

    @pl.when(c < C)
    def _encode():
        zp = jax.lax.dot_general(
            x_ref[...], we_ref[...],
            (((1,), (0,)), ((), ())),
            precision=jax.lax.Precision.DEFAULT,
            preferred_element_type=jnp.float32,
        ) + be_ref[...]
        zpre_ref[...] = zp
        zr = jnp.maximum(zp, 0.0)
        zscr_ref[c] = zr
        for g in range(GPC):
            m_ref[c * GPC + g] = jnp.max(zr[:, g * GW:(g + 1) * GW],
                                         axis=1, keepdims=True)

    @pl.when(c == C - 1)
    def _threshold():
        m = m_ref[...]
        lo0 = jax.lax.bitcast_convert_type(jnp.min(m, axis=0), jnp.int32)
        hi0 = jax.lax.bitcast_convert_type(jnp.max(m, axis=0), jnp.int32) + 1

        def cond(state):
            i, lo, hi = state
            return jnp.logical_and(i < 31,
                                   jnp.logical_not(jnp.all(hi - lo <= 1)))

        def body(state):
            i, lo, hi = state
            w3 = (hi - lo) // 3
            mid1 = lo + jnp.maximum(w3, 1)
            mid2 = jnp.maximum(lo + 2 * w3, mid1)
            m1f = jax.lax.bitcast_convert_type(mid1, jnp.float32)
            m2f = jax.lax.bitcast_convert_type(mid2, jnp.float32)
            p1, p2 = [], []
            for rb in range(BR // 128):
                rsl = slice(rb * 128, (rb + 1) * 128)
                m1rb = m1f[rsl]
                m2rb = m2f[rsl]
                a1 = jnp.zeros((128, 128), jnp.int32)
                a2 = jnp.zeros((128, 128), jnp.int32)
                for j in range(C):
                    zc = zscr_ref[j, rsl, :]
                    for k in range(BC // 128):
                        t = zc[:, k * 128:(k + 1) * 128]
                        a1 = a1 + (t >= m1rb).astype(jnp.int32)
                        a2 = a2 + (t >= m2rb).astype(jnp.int32)
                p1.append(jnp.sum(a1, axis=1)[:, None])
                p2.append(jnp.sum(a2, axis=1)[:, None])
            cnt1 = jnp.concatenate(p1, axis=0)
            cnt2 = jnp.concatenate(p2, axis=0)
            ge1 = cnt1 >= K
            ge2 = cnt2 >= K
            lo2 = jnp.where(ge2, mid2, jnp.where(ge1, mid1, lo))
            hi2 = jnp.where(ge2, hi, jnp.where(ge1, mid2, mid1))
            eq1 = cnt1 == K
            eq2 = cnt2 == K
            lo = jnp.where(eq1, mid1, jnp.where(eq2, mid2, lo2))
            hi = jnp.where(eq1, mid1, jnp.where(eq2, mid2, hi2))
            return i + 1, lo, hi

        _, lo, _ = jax.lax.while_loop(cond, body, (0, lo0, hi0))
        thr_ref[...] = jax.lax.bitcast_convert_type(lo, jnp.float32)

    @pl.when(c >= C)
    def _decode():
        j = c - C
        zc = zscr_ref[j]
        zs = jnp.where(zc >= thr_ref[...], zc, 0.0)
        zsp_ref[...] = zs
        part = jax.lax.dot_general(
            zs.astype(jnp.bfloat16), wd_ref[...],
            (((1,), (0,)), ((), ())),
            precision=jax.lax.Precision.DEFAULT,
            preferred_element_type=jnp.float32,
        )

        @pl.when(j == 0)
        def _():
            acc_ref[...] = part + bd_ref[...]

        @pl.when(j > 0)
        def _():
            acc_ref[...] = acc_ref[...] + part

        @pl.when(j == C - 1)
        def _():
            xrec_ref[...] = acc_ref[...]


def _build_call(N, D, S, K, BR, BC, interpret=False):
    C = S // BC
    R = N // BR
    GPC = max(1, -(-K // C))
    assert C * GPC >= K and BC % GPC == 0
    body = functools.partial(_sae_kernel_body, C, BR, BC, K, GPC)
    grid = (R, 2 * C)

    in_specs = [
        pl.BlockSpec((BR, D), lambda r, c: (r, 0)),
        pl.BlockSpec((D, BC), lambda r, c: (0, jnp.minimum(c, C - 1))),
        pl.BlockSpec((1, BC), lambda r, c: (0, jnp.minimum(c, C - 1))),
        pl.BlockSpec((BC, D), lambda r, c: (jnp.maximum(c - C, 0), 0)),
        pl.BlockSpec((1, D), lambda r, c: (0, 0)),
    ]
    out_specs = [
        pl.BlockSpec((BR, BC), lambda r, c: (r, jnp.minimum(c, C - 1))),
        pl.BlockSpec((BR, BC), lambda r, c: (r, jnp.maximum(c - C, 0))),
        pl.BlockSpec((BR, D), lambda r, c: (r, 0)),
    ]
    out_shape = [
        jax.ShapeDtypeStruct((N, S), jnp.float32),
        jax.ShapeDtypeStruct((N, S), jnp.float32),
        jax.ShapeDtypeStruct((N, D), jnp.float32),
    ]
    scratch_shapes = [
        pltpu.VMEM((C, BR, BC), jnp.float32),
        pltpu.VMEM((BR, D), jnp.float32),
        pltpu.VMEM((BR, 1), jnp.float32),
        pltpu.VMEM((C * GPC, BR, 1), jnp.float32),
    ]
    return pl.pallas_call(
        body,
        grid=grid,
        in_specs=in_specs,
        out_specs=out_specs,
        out_shape=out_shape,
        scratch_shapes=scratch_shapes,
        compiler_params=pltpu.CompilerParams(
            dimension_semantics=("arbitrary", "arbitrary"),
            vmem_limit_bytes=112 * 1024 * 1024,
        ),
        interpret=interpret,
    )


def _pick_block(n, target):
    b = min(n, target)
    while n % b:
        b -= 1
    return b


def kernel(x, W_enc, b_enc, W_dec, b_dec, *, _interpret=False):
    N, D = x.shape
    S = W_enc.shape[1]
    K = 32
    BR = _pick_block(N, 512)
    BC = _pick_block(S, 1024)
    call = _build_call(N, D, S, K, BR, BC, interpret=_interpret)
    z_pre, z_sparse, x_recon = call(
        x.astype(jnp.bfloat16), W_enc.astype(jnp.bfloat16),
        b_enc.reshape(1, S), W_dec.astype(jnp.bfloat16), b_dec.reshape(1, D))
    return (x_recon, z_sparse, z_pre)


# speedup vs baseline: 1.1220x; 1.1220x over previous
import functools

import jax
import jax.numpy as jnp
from jax.experimental import pallas as pl
from jax.experimental.pallas import tpu as pltpu


def _sae_kernel_body(C, BR, BC, K, GPC,
                     x_ref, we_ref, be_ref, wd_ref, bd_ref,
                     zpre_ref, zsp_ref, xrec_ref,
                     zscr_ref, acc_ref, thr_ref, m_ref):
    c = pl.program_id(1)
    GW = BC // GPC

    @pl.when(c < C)
    def _encode():
        zp = jax.lax.dot_general(
            x_ref[...], we_ref[...],
            (((1,), (0,)), ((), ())),
            precision=jax.lax.Precision.DEFAULT,
            preferred_element_type=jnp.float32,
        ) + be_ref[...]
        zpre_ref[...] = zp
        zr = jnp.maximum(zp, 0.0)
        zscr_ref[c] = zr
        for g in range(GPC):
            m_ref[c * GPC + g] = jnp.max(zr[:, g * GW:(g + 1) * GW],
                                         axis=1, keepdims=True)

    @pl.when(c == C - 1)
    def _threshold():
        m = m_ref[...]
        lo0 = jax.lax.bitcast_convert_type(jnp.min(m, axis=0), jnp.int32)
        hi0 = jax.lax.bitcast_convert_type(jnp.max(m, axis=0), jnp.int32) + 1

        def cond(state):
            i, lo, hi = state
            return jnp.logical_and(i < 31,
                                   jnp.logical_not(jnp.all(hi - lo <= 1)))

        def body(state):
            i, lo, hi = state
            mid = lo + (hi - lo) // 2
            midf = jax.lax.bitcast_convert_type(mid, jnp.float32)
            parts = []
            for rb in range(BR // 128):
                rsl = slice(rb * 128, (rb + 1) * 128)
                mrb = midf[rsl]
                acc = jnp.zeros((128, 128), jnp.int32)
                for j in range(C):
                    zc = zscr_ref[j, rsl, :]
                    for k in range(BC // 128):
                        acc = acc + (zc[:, k * 128:(k + 1) * 128]
                                     >= mrb).astype(jnp.int32)
                parts.append(jnp.sum(acc, axis=1)[:, None])
            cnt = jnp.concatenate(parts, axis=0)
            ge = cnt >= K
            lo = jnp.where(ge, mid, lo)
            hi = jnp.where(cnt == K, mid, jnp.where(ge, hi, mid))
            return i + 1, lo, hi

        _, lo, _ = jax.lax.while_loop(cond, body, (0, lo0, hi0))
        thr_ref[...] = jax.lax.bitcast_convert_type(lo, jnp.float32)

    @pl.when(c >= C)
    def _decode():
        j = c - C
        zc = zscr_ref[j]
        zs = jnp.where(zc >= thr_ref[...], zc, 0.0)
        zsp_ref[...] = zs
        part = jax.lax.dot_general(
            zs.astype(jnp.bfloat16), wd_ref[...],
            (((1,), (0,)), ((), ())),
            precision=jax.lax.Precision.DEFAULT,
            preferred_element_type=jnp.float32,
        )

        @pl.when(j == 0)
        def _():
            acc_ref[...] = part + bd_ref[...]

        @pl.when(j > 0)
        def _():
            acc_ref[...] = acc_ref[...] + part

        @pl.when(j == C - 1)
        def _():
            xrec_ref[...] = acc_ref[...]


def _build_call(N, D, S, K, BR, BC, interpret=False):
    C = S // BC
    R = N // BR
    GPC = max(1, -(-K // C))
    assert C * GPC >= K and BC % GPC == 0
    body = functools.partial(_sae_kernel_body, C, BR, BC, K, GPC)
    grid = (R, 2 * C)

    in_specs = [
        pl.BlockSpec((BR, D), lambda r, c: (r, 0)),
        pl.BlockSpec((D, BC), lambda r, c: (0, jnp.minimum(c, C - 1))),
        pl.BlockSpec((1, BC), lambda r, c: (0, jnp.minimum(c, C - 1))),
        pl.BlockSpec((BC, D), lambda r, c: (jnp.maximum(c - C, 0), 0)),
        pl.BlockSpec((1, D), lambda r, c: (0, 0)),
    ]
    out_specs = [
        pl.BlockSpec((BR, BC), lambda r, c: (r, jnp.minimum(c, C - 1))),
        pl.BlockSpec((BR, BC), lambda r, c: (r, jnp.maximum(c - C, 0))),
        pl.BlockSpec((BR, D), lambda r, c: (r, 0)),
    ]
    out_shape = [
        jax.ShapeDtypeStruct((N, S), jnp.float32),
        jax.ShapeDtypeStruct((N, S), jnp.float32),
        jax.ShapeDtypeStruct((N, D), jnp.float32),
    ]
    scratch_shapes = [
        pltpu.VMEM((C, BR, BC), jnp.float32),
        pltpu.VMEM((BR, D), jnp.float32),
        pltpu.VMEM((BR, 1), jnp.float32),
        pltpu.VMEM((C * GPC, BR, 1), jnp.float32),
    ]
    return pl.pallas_call(
        body,
        grid=grid,
        in_specs=in_specs,
        out_specs=out_specs,
        out_shape=out_shape,
        scratch_shapes=scratch_shapes,
        compiler_params=pltpu.CompilerParams(
            dimension_semantics=("arbitrary", "arbitrary"),
            vmem_limit_bytes=112 * 1024 * 1024,
        ),
        interpret=interpret,
    )


def _pick_block(n, target):
    b = min(n, target)
    while n % b:
        b -= 1
    return b


def kernel(x, W_enc, b_enc, W_dec, b_dec, *, _interpret=False):
    N, D = x.shape
    S = W_enc.shape[1]
    K = 32
    BR = _pick_block(N, 512)
    BC = _pick_block(S, 1024)
    call = _build_call(N, D, S, K, BR, BC, interpret=_interpret)
    z_pre, z_sparse, x_recon = call(
        x.astype(jnp.bfloat16), W_enc.astype(jnp.bfloat16),
        b_enc.reshape(1, S), W_dec.astype(jnp.bfloat16), b_dec.reshape(1, D))
    return (x_recon, z_sparse, z_pre)
